# Initial kernel scaffold; baseline (speedup 1.0000x reference)
import functools

import jax
import jax.numpy as jnp
from jax import lax
from jax.experimental import pallas as pl
from jax.experimental.pallas import tpu as pltpu
from jax.experimental.pallas import tpu_sc as plsc

N = 10000
E = 320000
D = 128

NC = 2
NS = 16
NW = NC * NS
EPT = E // NW
CHUNK = 80
NCH = EPT // CHUNK
NBUF = 5
ZROWS = N // NS

_mesh = plsc.VectorSubcoreMesh(
    core_axis_name="c", subcore_axis_name="s", num_cores=NC, num_subcores=NS
)


@functools.partial(
    pl.kernel,
    out_type=jax.ShapeDtypeStruct((NC, N, D), jnp.float32),
    mesh=_mesh,
    scratch_types=[
        pltpu.VMEM((NBUF * CHUNK + ZROWS, D), jnp.float32),
        pltpu.VMEM((EPT,), jnp.int32),
        pltpu.VMEM((NCH, CHUNK), jnp.int32),
        pltpu.VMEM_SHARED((N, D), jnp.float32),
        pltpu.SemaphoreType.DMA((NBUF,)),
        pltpu.SemaphoreType.DMA((NBUF,)),
    ],
)
def _sc_segsum(y_hbm, src_hbm, dst_hbm, out_hbm, rows, srcv, dstv, acc, gsem, ssem):
    c = lax.axis_index("c")
    s = lax.axis_index("s")
    w = c * NS + s

    zbase = NBUF * CHUNK

    @pl.loop(0, ZROWS)
    def _(i):
        @pl.loop(0, D // 16)
        def _(k):
            rows[zbase + i, pl.ds(k * 16, 16)] = jnp.zeros((16,), jnp.float32)

    pltpu.sync_copy(rows.at[pl.ds(zbase, ZROWS)], acc.at[pl.ds(s * ZROWS, ZROWS)])
    plsc.subcore_barrier()

    pltpu.sync_copy(src_hbm.at[pl.ds(w * EPT, EPT)], srcv)
    pltpu.sync_copy(dst_hbm.at[pl.ds(w * NCH, NCH)], dstv)

    for b in range(2):
        pltpu.async_copy(
            y_hbm.at[srcv.at[pl.ds(b * CHUNK, CHUNK)]],
            rows.at[pl.ds(b * CHUNK, CHUNK)],
            gsem.at[b],
        )

    @pl.loop(0, NCH // NBUF)
    def _(g):
        for b in range(NBUF):
            j = g * NBUF + b
            pltpu.make_async_copy(
                y_hbm.at[srcv.at[pl.ds(0, CHUNK)]],
                rows.at[pl.ds(b * CHUNK, CHUNK)],
                gsem.at[b],
            ).wait()
            pltpu.async_copy(
                rows.at[pl.ds(b * CHUNK, CHUNK)],
                acc.at[dstv.at[j]],
                ssem.at[b],
                add=True,
            )
            b2 = (b + 2) % NBUF

            @pl.when(j >= 3)
            def _():
                pltpu.make_async_copy(
                    rows.at[pl.ds(b2 * CHUNK, CHUNK)],
                    acc.at[pl.ds(0, CHUNK)],
                    ssem.at[b2],
                ).wait()

            @pl.when(j + 2 < NCH)
            def _():
                pltpu.async_copy(
                    y_hbm.at[srcv.at[pl.ds((j + 2) * CHUNK, CHUNK)]],
                    rows.at[pl.ds(b2 * CHUNK, CHUNK)],
                    gsem.at[b2],
                )

    for b in range(NBUF):
        pltpu.make_async_copy(
            rows.at[pl.ds(b * CHUNK, CHUNK)],
            acc.at[pl.ds(0, CHUNK)],
            ssem.at[b],
        ).wait()

    plsc.subcore_barrier()

    pltpu.sync_copy(
        acc.at[pl.ds(s * ZROWS, ZROWS)],
        out_hbm.at[c, pl.ds(s * ZROWS, ZROWS)],
    )


_BLK = 1000


def _dot(a, w):
    return lax.dot_general(
        a, w, (((1,), (0,)), ((), ())),
        precision=lax.Precision.HIGHEST,
        preferred_element_type=jnp.float32,
    )


def _pre_body(x_ref, wa_ref, wb_ref, ya_ref, yb_ref):
    xb = x_ref[...]
    ya_ref[...] = _dot(xb, wa_ref[...])
    yb_ref[...] = _dot(xb, wb_ref[...])


def _mid_body(p0_ref, p1_ref, r_ref, b_ref, wa_ref, wb_ref, ya_ref, yb_ref):
    h = jnp.maximum(p0_ref[...] + p1_ref[...] + r_ref[...] + b_ref[...], 0.0)
    ya_ref[...] = _dot(h, wa_ref[...])
    yb_ref[...] = _dot(h, wb_ref[...])


def _post_body(p0_ref, p1_ref, r_ref, b_ref, o_ref):
    o_ref[...] = p0_ref[...] + p1_ref[...] + r_ref[...] + b_ref[...]


_row_spec = pl.BlockSpec((_BLK, D), lambda i: (i, 0))
_w_spec = pl.BlockSpec((D, D), lambda i: (0, 0))
_b_spec = pl.BlockSpec((1, D), lambda i: (0, 0))
_f32 = jnp.float32


def _pre(x, wa, wb):
    return pl.pallas_call(
        _pre_body,
        grid=(N // _BLK,),
        in_specs=[_row_spec, _w_spec, _w_spec],
        out_specs=[_row_spec, _row_spec],
        out_shape=[jax.ShapeDtypeStruct((N, D), _f32)] * 2,
    )(x, wa, wb)


def _mid(p0, p1, r, b, wa, wb):
    return pl.pallas_call(
        _mid_body,
        grid=(N // _BLK,),
        in_specs=[_row_spec, _row_spec, _row_spec, _b_spec, _w_spec, _w_spec],
        out_specs=[_row_spec, _row_spec],
        out_shape=[jax.ShapeDtypeStruct((N, D), _f32)] * 2,
    )(p0, p1, r, b, wa, wb)


def _post(p0, p1, r, b):
    return pl.pallas_call(
        _post_body,
        grid=(N // _BLK,),
        in_specs=[_row_spec, _row_spec, _row_spec, _b_spec],
        out_specs=_row_spec,
        out_shape=jax.ShapeDtypeStruct((N, D), _f32),
    )(p0, p1, r, b)


def kernel(x, edge_index, W1_rel, b1, W1_root, W2_rel, b2, W2_root):
    src = edge_index[0]
    dst2d = edge_index[1].reshape(E // CHUNK, CHUNK)
    b1r = b1.reshape(1, D)
    b2r = b2.reshape(1, D)

    y1, r1 = _pre(x, W1_rel, W1_root)
    p1 = _sc_segsum(y1, src, dst2d)
    y2, r2 = _mid(p1[0], p1[1], r1, b1r, W2_rel, W2_root)
    p2 = _sc_segsum(y2, src, dst2d)
    return _post(p2[0], p2[1], r2, b2r)


# speedup vs baseline: 6.6717x; 6.6717x over previous
import functools

import jax
import jax.numpy as jnp
from jax import lax
from jax.experimental import pallas as pl
from jax.experimental.pallas import tpu as pltpu
from jax.experimental.pallas import tpu_sc as plsc

N = 10000
E = 320000
D = 128

NC = 2
NS = 16
NW = NC * NS
SEG = 2000
CHUNK = 40
NCH = SEG // CHUNK
NBUF = 5
RING = NBUF * CHUNK
NP = 10240
WPT = NP // NS
HB = 80

_mesh = plsc.VectorSubcoreMesh(
    core_axis_name="c", subcore_axis_name="s", num_cores=NC, num_subcores=NS
)


def _zero_ring(rows):
    @pl.loop(0, RING)
    def _(i):
        @pl.loop(0, D // 16)
        def _(k):
            rows[i, pl.ds(k * 16, 16)] = jnp.zeros((16,), jnp.float32)


def _zero_acc_slice(rows, acc, s):
    @pl.loop(0, 3)
    def _(z):
        pltpu.sync_copy(
            rows.at[pl.ds(0, RING)], acc.at[pl.ds(s * WPT + z * RING, RING)]
        )

    pltpu.sync_copy(
        rows.at[pl.ds(0, WPT - 3 * RING)],
        acc.at[pl.ds(s * WPT + 3 * RING, WPT - 3 * RING)],
    )


def _stage(table, src_hbm, src_off, dst_ref, rows, srcv, dstv, acc, gsem, ssem):
    pltpu.sync_copy(src_hbm.at[pl.ds(src_off, SEG)], srcv)
    pltpu.sync_copy(dst_ref, dstv)

    for b in range(2):
        pltpu.async_copy(
            table.at[srcv.at[pl.ds(b * CHUNK, CHUNK)]],
            rows.at[pl.ds(b * CHUNK, CHUNK)],
            gsem.at[b],
        )

    @pl.loop(0, NCH // NBUF)
    def _(g):
        for b in range(NBUF):
            j = g * NBUF + b
            pltpu.make_async_copy(
                table.at[srcv.at[pl.ds(0, CHUNK)]],
                rows.at[pl.ds(b * CHUNK, CHUNK)],
                gsem.at[b],
            ).wait()
            pltpu.async_copy(
                rows.at[pl.ds(b * CHUNK, CHUNK)],
                acc.at[dstv.at[j, 0]],
                ssem.at[b],
                add=True,
            )
            b2 = (b + 2) % NBUF

            @pl.when((j >= 3) & (j + 2 < NCH))
            def _():
                pltpu.make_async_copy(
                    rows.at[pl.ds(b2 * CHUNK, CHUNK)],
                    acc.at[pl.ds(0, CHUNK)],
                    ssem.at[b2],
                ).wait()

            @pl.when(j + 2 < NCH)
            def _():
                pltpu.async_copy(
                    table.at[srcv.at[pl.ds((j + 2) * CHUNK, CHUNK)]],
                    rows.at[pl.ds(b2 * CHUNK, CHUNK)],
                    gsem.at[b2],
                )

    for b in range(NBUF):
        pltpu.make_async_copy(
            rows.at[pl.ds(b * CHUNK, CHUNK)],
            acc.at[pl.ds(0, CHUNK)],
            ssem.at[b],
        ).wait()


@functools.partial(
    pl.kernel,
    out_type=(
        jax.ShapeDtypeStruct((NC, NP, D), jnp.float32),
        jax.ShapeDtypeStruct((NC, NP, D), jnp.float32),
    ),
    mesh=_mesh,
    scratch_types=[
        pltpu.VMEM((RING, D), jnp.float32),
        pltpu.VMEM((SEG,), jnp.int32),
        pltpu.VMEM((NCH, 1, CHUNK), jnp.int32),
        pltpu.VMEM_SHARED((NP, D), jnp.float32),
        pltpu.SemaphoreType.DMA((NBUF,)),
        pltpu.SemaphoreType.DMA((NBUF,)),
    ],
)
def _sc_gnn(y_hbm, rb_hbm, src_hbm, dstA, dstB, h_hbm, p2_hbm,
            rows, srcv, dstv, acc, gsem, ssem):
    c = lax.axis_index("c")
    s = lax.axis_index("s")
    w = c * NS + s

    _zero_ring(rows)
    _zero_acc_slice(rows, acc, s)
    plsc.subcore_barrier()

    @pl.loop(0, 10)
    def _(st):
        _stage(y_hbm, src_hbm, s * (10 * SEG) + st * SEG, dstA.at[s, st],
               rows, srcv, dstv, acc, gsem, ssem)

    plsc.subcore_barrier()

    @pl.loop(0, WPT // HB)
    def _(blk):
        base = s * WPT + blk * HB
        pltpu.sync_copy(acc.at[pl.ds(base, HB)], rows.at[pl.ds(0, HB)])
        pltpu.sync_copy(rb_hbm.at[pl.ds(base, HB)], rows.at[pl.ds(HB, HB)])

        @pl.loop(0, HB)
        def _(i):
            @pl.loop(0, D // 16)
            def _(k):
                sl = pl.ds(k * 16, 16)
                rows[i, sl] = jnp.maximum(rows[i, sl] + rows[HB + i, sl], 0.0)

        pltpu.sync_copy(rows.at[pl.ds(0, HB)], h_hbm.at[c, pl.ds(base, HB)])

    _zero_ring(rows)
    _zero_acc_slice(rows, acc, s)
    plsc.subcore_barrier()

    @pl.loop(0, 5)
    def _(st):
        _stage(h_hbm.at[c], src_hbm, w * (5 * SEG) + st * SEG, dstB.at[w, st],
               rows, srcv, dstv, acc, gsem, ssem)
    plsc.subcore_barrier()

    @pl.loop(0, 3)
    def _(z):
        base = s * WPT + z * RING
        pltpu.sync_copy(acc.at[pl.ds(base, RING)], rows.at[pl.ds(0, RING)])
        pltpu.sync_copy(rows.at[pl.ds(0, RING)], p2_hbm.at[c, pl.ds(base, RING)])

    base = s * WPT + 3 * RING
    rem = WPT - 3 * RING
    pltpu.sync_copy(acc.at[pl.ds(base, rem)], rows.at[pl.ds(0, rem)])
    pltpu.sync_copy(rows.at[pl.ds(0, rem)], p2_hbm.at[c, pl.ds(base, rem)])


_BLK = 640


def _dot(a, w):
    return lax.dot_general(
        a, w, (((1,), (0,)), ((), ())),
        precision=lax.Precision.HIGHEST,
        preferred_element_type=jnp.float32,
    )


def _pre_body(x_ref, wa_ref, wb_ref, b_ref, y_ref, rb_ref):
    xb = x_ref[...]
    y_ref[...] = _dot(xb, wa_ref[...])
    rb_ref[...] = _dot(xb, wb_ref[...]) + b_ref[...]


def _post_body(p_ref, h_ref, wa_ref, wb_ref, b_ref, o_ref):
    agg = p_ref[0] + p_ref[1]
    o_ref[...] = _dot(agg, wa_ref[...]) + _dot(h_ref[...], wb_ref[...]) + b_ref[...]


_row_spec = pl.BlockSpec((_BLK, D), lambda i: (i, 0))
_p_spec = pl.BlockSpec((NC, _BLK, D), lambda i: (0, i, 0))
_h_spec = pl.BlockSpec((1, _BLK, D), lambda i: (0, i, 0))
_w_spec = pl.BlockSpec((D, D), lambda i: (0, 0))
_b_spec = pl.BlockSpec((1, D), lambda i: (0, 0))
_f32 = jnp.float32


def _pre(x, wa, wb, b):
    return pl.pallas_call(
        _pre_body,
        grid=(NP // _BLK,),
        in_specs=[_row_spec, _w_spec, _w_spec, _b_spec],
        out_specs=[_row_spec, _row_spec],
        out_shape=[jax.ShapeDtypeStruct((NP, D), _f32)] * 2,
    )(x, wa, wb, b)


def _post(p, h, wa, wb, b):
    def h2d_body(p_ref, h_ref, wa_ref, wb_ref, b_ref, o_ref):
        _post_body(p_ref, h_ref[0], wa_ref, wb_ref, b_ref, o_ref)

    return pl.pallas_call(
        h2d_body,
        grid=(NP // _BLK,),
        in_specs=[_p_spec, _h_spec, _w_spec, _w_spec, _b_spec],
        out_specs=_row_spec,
        out_shape=jax.ShapeDtypeStruct((N, D), _f32),
    )(p, h, wa, wb, b)


def kernel(x, edge_index, W1_rel, b1, W1_root, W2_rel, b2, W2_root):
    src = edge_index[0]
    dst = edge_index[1]
    dstA = dst.reshape(NS, 10, NCH, 1, CHUNK)
    dstB = dst.reshape(NW, 5, NCH, 1, CHUNK)
    b1r = b1.reshape(1, D)
    b2r = b2.reshape(1, D)

    y1, r1b = _pre(x, W1_rel, W1_root, b1r)
    h, p2 = _sc_gnn(y1, r1b, src, dstA, dstB)
    return _post(p2, h, W2_rel, W2_root, b2r)


# speedup vs baseline: 6.8569x; 1.0278x over previous
import functools

import jax
import jax.numpy as jnp
from jax import lax
from jax.experimental import pallas as pl
from jax.experimental.pallas import tpu as pltpu
from jax.experimental.pallas import tpu_sc as plsc

N = 10000
E = 320000
D = 128

NC = 2
NS = 16
NW = NC * NS
SEG = 5000
CHUNK = 40
NCH = SEG // CHUNK
NBUF = 5
RING = NBUF * CHUNK
NP = 10240
WPT = NP // NS
HB = 80

_mesh = plsc.VectorSubcoreMesh(
    core_axis_name="c", subcore_axis_name="s", num_cores=NC, num_subcores=NS
)


def _zero_ring(rows):
    @pl.loop(0, RING)
    def _(i):
        @pl.loop(0, D // 16)
        def _(k):
            rows[i, pl.ds(k * 16, 16)] = jnp.zeros((16,), jnp.float32)


_PIECES = [
    (o, min(RING, WPT - o)) for o in range(0, WPT, RING)
]


def _zero_acc_slice(rows, acc, s):
    for off, sz in _PIECES:
        pltpu.sync_copy(
            rows.at[pl.ds(0, sz)], acc.at[pl.ds(s * WPT + off, sz)]
        )


def _stage(table, src_hbm, src_off, dst_ref, rows, srcv, dstv, acc, gsem, ssem):
    pltpu.sync_copy(src_hbm.at[pl.ds(src_off, SEG)], srcv)
    pltpu.sync_copy(dst_ref, dstv)

    for b in range(2):
        pltpu.async_copy(
            table.at[srcv.at[pl.ds(b * CHUNK, CHUNK)]],
            rows.at[pl.ds(b * CHUNK, CHUNK)],
            gsem.at[b],
        )

    @pl.loop(0, NCH // NBUF)
    def _(g):
        for b in range(NBUF):
            j = g * NBUF + b
            pltpu.make_async_copy(
                table.at[srcv.at[pl.ds(0, CHUNK)]],
                rows.at[pl.ds(b * CHUNK, CHUNK)],
                gsem.at[b],
            ).wait()
            pltpu.async_copy(
                rows.at[pl.ds(b * CHUNK, CHUNK)],
                acc.at[dstv.at[j, 0]],
                ssem.at[b],
                add=True,
            )
            b2 = (b + 2) % NBUF

            @pl.when((j >= 3) & (j + 2 < NCH))
            def _():
                pltpu.make_async_copy(
                    rows.at[pl.ds(b2 * CHUNK, CHUNK)],
                    acc.at[pl.ds(0, CHUNK)],
                    ssem.at[b2],
                ).wait()

            @pl.when(j + 2 < NCH)
            def _():
                pltpu.async_copy(
                    table.at[srcv.at[pl.ds((j + 2) * CHUNK, CHUNK)]],
                    rows.at[pl.ds(b2 * CHUNK, CHUNK)],
                    gsem.at[b2],
                )

    for b in range(NBUF):
        pltpu.make_async_copy(
            rows.at[pl.ds(b * CHUNK, CHUNK)],
            acc.at[pl.ds(0, CHUNK)],
            ssem.at[b],
        ).wait()


@functools.partial(
    pl.kernel,
    out_type=(
        jax.ShapeDtypeStruct((NC, NP, D), jnp.float32),
        jax.ShapeDtypeStruct((NC, NP, D), jnp.float32),
    ),
    mesh=_mesh,
    scratch_types=[
        pltpu.VMEM((RING, D), jnp.float32),
        pltpu.VMEM((SEG,), jnp.int32),
        pltpu.VMEM((NCH, 1, CHUNK), jnp.int32),
        pltpu.VMEM_SHARED((NP, D), jnp.float32),
        pltpu.SemaphoreType.DMA((NBUF,)),
        pltpu.SemaphoreType.DMA((NBUF,)),
    ],
)
def _sc_gnn(y_hbm, rb_hbm, src_hbm, dstA, dstB, h_hbm, p2_hbm,
            rows, srcv, dstv, acc, gsem, ssem):
    c = lax.axis_index("c")
    s = lax.axis_index("s")
    w = c * NS + s

    _zero_ring(rows)
    _zero_acc_slice(rows, acc, s)
    plsc.subcore_barrier()

    @pl.loop(0, 4)
    def _(st):
        _stage(y_hbm, src_hbm, s * (4 * SEG) + st * SEG, dstA.at[s, st],
               rows, srcv, dstv, acc, gsem, ssem)

    plsc.subcore_barrier()

    @pl.loop(0, WPT // HB)
    def _(blk):
        base = s * WPT + blk * HB
        pltpu.sync_copy(acc.at[pl.ds(base, HB)], rows.at[pl.ds(0, HB)])
        pltpu.sync_copy(rb_hbm.at[pl.ds(base, HB)], rows.at[pl.ds(HB, HB)])

        @pl.loop(0, HB)
        def _(i):
            @pl.loop(0, D // 16)
            def _(k):
                sl = pl.ds(k * 16, 16)
                rows[i, sl] = jnp.maximum(rows[i, sl] + rows[HB + i, sl], 0.0)

        pltpu.sync_copy(rows.at[pl.ds(0, HB)], h_hbm.at[c, pl.ds(base, HB)])

    _zero_ring(rows)
    _zero_acc_slice(rows, acc, s)
    plsc.subcore_barrier()

    @pl.loop(0, 2)
    def _(st):
        _stage(h_hbm.at[c], src_hbm, w * (2 * SEG) + st * SEG, dstB.at[w, st],
               rows, srcv, dstv, acc, gsem, ssem)
    plsc.subcore_barrier()

    for off, sz in _PIECES:
        base = s * WPT + off
        pltpu.sync_copy(acc.at[pl.ds(base, sz)], rows.at[pl.ds(0, sz)])
        pltpu.sync_copy(rows.at[pl.ds(0, sz)], p2_hbm.at[c, pl.ds(base, sz)])


_BLK = 640


def _dot(a, w):
    return lax.dot_general(
        a, w, (((1,), (0,)), ((), ())),
        precision=lax.Precision.HIGHEST,
        preferred_element_type=jnp.float32,
    )


def _pre_body(x_ref, wa_ref, wb_ref, b_ref, y_ref, rb_ref):
    xb = x_ref[...]
    y_ref[...] = _dot(xb, wa_ref[...])
    rb_ref[...] = _dot(xb, wb_ref[...]) + b_ref[...]


def _post_body(p_ref, h_ref, wa_ref, wb_ref, b_ref, o_ref):
    agg = p_ref[0] + p_ref[1]
    o_ref[...] = _dot(agg, wa_ref[...]) + _dot(h_ref[...], wb_ref[...]) + b_ref[...]


_row_spec = pl.BlockSpec((_BLK, D), lambda i: (i, 0))
_p_spec = pl.BlockSpec((NC, _BLK, D), lambda i: (0, i, 0))
_h_spec = pl.BlockSpec((1, _BLK, D), lambda i: (0, i, 0))
_w_spec = pl.BlockSpec((D, D), lambda i: (0, 0))
_b_spec = pl.BlockSpec((1, D), lambda i: (0, 0))
_f32 = jnp.float32


def _pre(x, wa, wb, b):
    return pl.pallas_call(
        _pre_body,
        grid=(NP // _BLK,),
        in_specs=[_row_spec, _w_spec, _w_spec, _b_spec],
        out_specs=[_row_spec, _row_spec],
        out_shape=[jax.ShapeDtypeStruct((NP, D), _f32)] * 2,
    )(x, wa, wb, b)


def _post(p, h, wa, wb, b):
    def h2d_body(p_ref, h_ref, wa_ref, wb_ref, b_ref, o_ref):
        _post_body(p_ref, h_ref[0], wa_ref, wb_ref, b_ref, o_ref)

    return pl.pallas_call(
        h2d_body,
        grid=(NP // _BLK,),
        in_specs=[_p_spec, _h_spec, _w_spec, _w_spec, _b_spec],
        out_specs=_row_spec,
        out_shape=jax.ShapeDtypeStruct((N, D), _f32),
    )(p, h, wa, wb, b)


def kernel(x, edge_index, W1_rel, b1, W1_root, W2_rel, b2, W2_root):
    src = edge_index[0]
    dst = edge_index[1]
    dstA = dst.reshape(NS, 4, NCH, 1, CHUNK)
    dstB = dst.reshape(NW, 2, NCH, 1, CHUNK)
    b1r = b1.reshape(1, D)
    b2r = b2.reshape(1, D)

    y1, r1b = _pre(x, W1_rel, W1_root, b1r)
    h, p2 = _sc_gnn(y1, r1b, src, dstA, dstB)
    return _post(p2, h, W2_rel, W2_root, b2r)


# speedup vs baseline: 8.3642x; 1.2198x over previous
import functools

import jax
import jax.numpy as jnp
from jax import lax
from jax.experimental import pallas as pl
from jax.experimental.pallas import tpu as pltpu
from jax.experimental.pallas import tpu_sc as plsc

N = 10000
E = 320000
D = 128

NC = 2
NS = 16
NW = NC * NS
SEG = 5000
CHUNK = 40
NCH = SEG // CHUNK
NBUF = 5
RING = NBUF * CHUNK
NP = 10240
WPT = NP // NS
HB = 80

_mesh = plsc.VectorSubcoreMesh(
    core_axis_name="c", subcore_axis_name="s", num_cores=NC, num_subcores=NS
)


def _zero_ring(rows):
    @pl.loop(0, RING)
    def _(i):
        @pl.loop(0, D // 16)
        def _(k):
            rows[i, pl.ds(k * 16, 16)] = jnp.zeros((16,), jnp.float32)


_PIECES = [
    (o, min(RING, WPT - o)) for o in range(0, WPT, RING)
]


def _zero_acc_slice(rows, acc, s):
    for off, sz in _PIECES:
        pltpu.sync_copy(
            rows.at[pl.ds(0, sz)], acc.at[pl.ds(s * WPT + off, sz)]
        )


def _stage(table, src_hbm, src_off, dst_ref, rows, srcv, dstv, acc, gsem, ssem):
    pltpu.sync_copy(src_hbm.at[pl.ds(src_off, SEG)], srcv)
    pltpu.sync_copy(dst_ref, dstv)

    for b in range(3):
        pltpu.async_copy(
            table.at[srcv.at[pl.ds(b * CHUNK, CHUNK)]],
            rows.at[pl.ds(b * CHUNK, CHUNK)],
            gsem.at[b],
        )

    @pl.loop(0, NCH // NBUF)
    def _(g):
        for b in range(NBUF):
            j = g * NBUF + b
            pltpu.make_async_copy(
                table.at[srcv.at[pl.ds(0, CHUNK)]],
                rows.at[pl.ds(b * CHUNK, CHUNK)],
                gsem.at[b],
            ).wait()
            pltpu.async_copy(
                rows.at[pl.ds(b * CHUNK, CHUNK)],
                acc.at[dstv.at[j, 0]],
                ssem.at[b],
                add=True,
            )
            b2 = (b + 3) % NBUF

            @pl.when((j >= 2) & (j + 3 < NCH))
            def _():
                pltpu.make_async_copy(
                    rows.at[pl.ds(b2 * CHUNK, CHUNK)],
                    acc.at[pl.ds(0, CHUNK)],
                    ssem.at[b2],
                ).wait()

            @pl.when(j + 3 < NCH)
            def _():
                pltpu.async_copy(
                    table.at[srcv.at[pl.ds((j + 3) * CHUNK, CHUNK)]],
                    rows.at[pl.ds(b2 * CHUNK, CHUNK)],
                    gsem.at[b2],
                )

    for b in range(NBUF):
        pltpu.make_async_copy(
            rows.at[pl.ds(b * CHUNK, CHUNK)],
            acc.at[pl.ds(0, CHUNK)],
            ssem.at[b],
        ).wait()


@functools.partial(
    pl.kernel,
    out_type=(
        jax.ShapeDtypeStruct((NC, NP, D), jnp.float32),
        jax.ShapeDtypeStruct((NC, NP, D), jnp.float32),
    ),
    mesh=_mesh,
    scratch_types=[
        pltpu.VMEM((RING, D), jnp.float32),
        pltpu.VMEM((SEG,), jnp.int32),
        pltpu.VMEM((NCH, 1, CHUNK), jnp.int32),
        pltpu.VMEM_SHARED((NP, D), jnp.float32),
        pltpu.SemaphoreType.DMA((NBUF,)),
        pltpu.SemaphoreType.DMA((NBUF,)),
    ],
)
def _sc_gnn(y_hbm, rb_hbm, src_hbm, dstA, dstB, h_hbm, p2_hbm,
            rows, srcv, dstv, acc, gsem, ssem):
    c = lax.axis_index("c")
    s = lax.axis_index("s")
    w = c * NS + s

    _zero_ring(rows)
    _zero_acc_slice(rows, acc, s)
    plsc.subcore_barrier()

    @pl.loop(0, 4)
    def _(st):
        _stage(y_hbm, src_hbm, s * (4 * SEG) + st * SEG, dstA.at[s, st],
               rows, srcv, dstv, acc, gsem, ssem)

    plsc.subcore_barrier()

    @pl.loop(0, WPT // HB)
    def _(blk):
        base = s * WPT + blk * HB
        pltpu.sync_copy(acc.at[pl.ds(base, HB)], rows.at[pl.ds(0, HB)])
        pltpu.sync_copy(rb_hbm.at[pl.ds(base, HB)], rows.at[pl.ds(HB, HB)])

        @pl.loop(0, HB)
        def _(i):
            @pl.loop(0, D // 16)
            def _(k):
                sl = pl.ds(k * 16, 16)
                rows[i, sl] = jnp.maximum(rows[i, sl] + rows[HB + i, sl], 0.0)

        pltpu.sync_copy(rows.at[pl.ds(0, HB)], h_hbm.at[c, pl.ds(base, HB)])

    _zero_ring(rows)
    _zero_acc_slice(rows, acc, s)
    plsc.subcore_barrier()

    @pl.loop(0, 2)
    def _(st):
        _stage(h_hbm.at[c], src_hbm, w * (2 * SEG) + st * SEG, dstB.at[w, st],
               rows, srcv, dstv, acc, gsem, ssem)
    plsc.subcore_barrier()

    for off, sz in _PIECES:
        base = s * WPT + off
        pltpu.sync_copy(acc.at[pl.ds(base, sz)], rows.at[pl.ds(0, sz)])
        pltpu.sync_copy(rows.at[pl.ds(0, sz)], p2_hbm.at[c, pl.ds(base, sz)])


_BLK = 640


def _dot(a, w):
    return lax.dot_general(
        a, w, (((1,), (0,)), ((), ())),
        precision=lax.Precision.HIGHEST,
        preferred_element_type=jnp.float32,
    )


def _pre_body(x_ref, wa_ref, wb_ref, b_ref, y_ref, rb_ref):
    xb = x_ref[...]
    y_ref[...] = _dot(xb, wa_ref[...])
    rb_ref[...] = _dot(xb, wb_ref[...]) + b_ref[...]


def _post_body(p_ref, h_ref, wa_ref, wb_ref, b_ref, o_ref):
    agg = p_ref[0] + p_ref[1]
    o_ref[...] = _dot(agg, wa_ref[...]) + _dot(h_ref[...], wb_ref[...]) + b_ref[...]


_row_spec = pl.BlockSpec((_BLK, D), lambda i: (i, 0))
_p_spec = pl.BlockSpec((NC, _BLK, D), lambda i: (0, i, 0))
_h_spec = pl.BlockSpec((1, _BLK, D), lambda i: (0, i, 0))
_w_spec = pl.BlockSpec((D, D), lambda i: (0, 0))
_b_spec = pl.BlockSpec((1, D), lambda i: (0, 0))
_f32 = jnp.float32


def _pre(x, wa, wb, b):
    return pl.pallas_call(
        _pre_body,
        grid=(NP // _BLK,),
        in_specs=[_row_spec, _w_spec, _w_spec, _b_spec],
        out_specs=[_row_spec, _row_spec],
        out_shape=[jax.ShapeDtypeStruct((NP, D), _f32)] * 2,
    )(x, wa, wb, b)


def _post(p, h, wa, wb, b):
    def h2d_body(p_ref, h_ref, wa_ref, wb_ref, b_ref, o_ref):
        _post_body(p_ref, h_ref[0], wa_ref, wb_ref, b_ref, o_ref)

    return pl.pallas_call(
        h2d_body,
        grid=(NP // _BLK,),
        in_specs=[_p_spec, _h_spec, _w_spec, _w_spec, _b_spec],
        out_specs=_row_spec,
        out_shape=jax.ShapeDtypeStruct((N, D), _f32),
    )(p, h, wa, wb, b)


def kernel(x, edge_index, W1_rel, b1, W1_root, W2_rel, b2, W2_root):
    src = edge_index[0]
    dst = edge_index[1]
    dstA = dst.reshape(NS, 4, NCH, 1, CHUNK)
    dstB = dst.reshape(NW, 2, NCH, 1, CHUNK)
    b1r = b1.reshape(1, D)
    b2r = b2.reshape(1, D)

    y1, r1b = _pre(x, W1_rel, W1_root, b1r)
    h, p2 = _sc_gnn(y1, r1b, src, dstA, dstB)
    return _post(p2, h, W2_rel, W2_root, b2r)


# speedup vs baseline: 9.0021x; 1.0763x over previous
import functools

import jax
import jax.numpy as jnp
from jax import lax
from jax.experimental import pallas as pl
from jax.experimental.pallas import tpu as pltpu
from jax.experimental.pallas import tpu_sc as plsc

N = 10000
E = 320000
D = 128

NC = 2
NS = 16
NW = NC * NS
SEG = 5000
CHUNK = 40
NCH = SEG // CHUNK
NBUF = 5
RING = NBUF * CHUNK
NP = 10240
WPT = NP // NS
HB = 80

_mesh = plsc.VectorSubcoreMesh(
    core_axis_name="c", subcore_axis_name="s", num_cores=NC, num_subcores=NS
)


def _zero_ring(rows):
    @pl.loop(0, RING)
    def _(i):
        @pl.loop(0, D // 16)
        def _(k):
            rows[i, pl.ds(k * 16, 16)] = jnp.zeros((16,), jnp.float32)


_PIECES = [
    (o, min(RING, WPT - o)) for o in range(0, WPT, RING)
]


def _zero_acc_slice(rows, acc, s):
    for off, sz in _PIECES:
        pltpu.sync_copy(
            rows.at[pl.ds(0, sz)], acc.at[pl.ds(s * WPT + off, sz)]
        )


def _stage(table, src_hbm, src_off, dst_ref, rows, srcv, dstv, acc, gsem, ssem):
    pltpu.sync_copy(src_hbm.at[pl.ds(src_off, SEG)], srcv)
    pltpu.sync_copy(dst_ref, dstv)

    for b in range(3):
        pltpu.async_copy(
            table.at[srcv.at[pl.ds(b * CHUNK, CHUNK)]],
            rows.at[pl.ds(b * CHUNK, CHUNK)],
            gsem.at[b],
        )

    @pl.loop(0, NCH // NBUF)
    def _(g):
        for b in range(NBUF):
            j = g * NBUF + b
            b2 = (b + 3) % NBUF

            @pl.when((j >= 2) & (j + 3 < NCH))
            def _():
                pltpu.make_async_copy(
                    rows.at[pl.ds(b2 * CHUNK, CHUNK)],
                    acc.at[pl.ds(0, CHUNK)],
                    ssem.at[b2],
                ).wait()

            @pl.when(j + 3 < NCH)
            def _():
                pltpu.async_copy(
                    table.at[srcv.at[pl.ds((j + 3) * CHUNK, CHUNK)]],
                    rows.at[pl.ds(b2 * CHUNK, CHUNK)],
                    gsem.at[b2],
                )

            pltpu.make_async_copy(
                table.at[srcv.at[pl.ds(0, CHUNK)]],
                rows.at[pl.ds(b * CHUNK, CHUNK)],
                gsem.at[b],
            ).wait()
            pltpu.async_copy(
                rows.at[pl.ds(b * CHUNK, CHUNK)],
                acc.at[dstv.at[j, 0]],
                ssem.at[b],
                add=True,
            )

    for b in range(NBUF):
        pltpu.make_async_copy(
            rows.at[pl.ds(b * CHUNK, CHUNK)],
            acc.at[pl.ds(0, CHUNK)],
            ssem.at[b],
        ).wait()


@functools.partial(
    pl.kernel,
    out_type=(
        jax.ShapeDtypeStruct((NC, NP, D), jnp.float32),
        jax.ShapeDtypeStruct((NC, NP, D), jnp.float32),
    ),
    mesh=_mesh,
    scratch_types=[
        pltpu.VMEM((RING, D), jnp.float32),
        pltpu.VMEM((SEG,), jnp.int32),
        pltpu.VMEM((NCH, 1, CHUNK), jnp.int32),
        pltpu.VMEM_SHARED((NP, D), jnp.float32),
        pltpu.SemaphoreType.DMA((NBUF,)),
        pltpu.SemaphoreType.DMA((NBUF,)),
    ],
)
def _sc_gnn(y_hbm, rb_hbm, src_hbm, dstA, dstB, h_hbm, p2_hbm,
            rows, srcv, dstv, acc, gsem, ssem):
    c = lax.axis_index("c")
    s = lax.axis_index("s")
    w = c * NS + s

    _zero_ring(rows)
    _zero_acc_slice(rows, acc, s)
    plsc.subcore_barrier()

    @pl.loop(0, 4)
    def _(st):
        _stage(y_hbm, src_hbm, s * (4 * SEG) + st * SEG, dstA.at[s, st],
               rows, srcv, dstv, acc, gsem, ssem)

    plsc.subcore_barrier()

    @pl.loop(0, WPT // HB)
    def _(blk):
        base = s * WPT + blk * HB
        pltpu.sync_copy(acc.at[pl.ds(base, HB)], rows.at[pl.ds(0, HB)])
        pltpu.sync_copy(rb_hbm.at[pl.ds(base, HB)], rows.at[pl.ds(HB, HB)])

        @pl.loop(0, HB)
        def _(i):
            @pl.loop(0, D // 16)
            def _(k):
                sl = pl.ds(k * 16, 16)
                rows[i, sl] = jnp.maximum(rows[i, sl] + rows[HB + i, sl], 0.0)

        pltpu.sync_copy(rows.at[pl.ds(0, HB)], h_hbm.at[c, pl.ds(base, HB)])

    _zero_ring(rows)
    _zero_acc_slice(rows, acc, s)
    plsc.subcore_barrier()

    @pl.loop(0, 2)
    def _(st):
        _stage(h_hbm.at[c], src_hbm, w * (2 * SEG) + st * SEG, dstB.at[w, st],
               rows, srcv, dstv, acc, gsem, ssem)
    plsc.subcore_barrier()

    for off, sz in _PIECES:
        base = s * WPT + off
        pltpu.sync_copy(acc.at[pl.ds(base, sz)], rows.at[pl.ds(0, sz)])
        pltpu.sync_copy(rows.at[pl.ds(0, sz)], p2_hbm.at[c, pl.ds(base, sz)])


_BLK = 640


def _dot(a, w):
    return lax.dot_general(
        a, w, (((1,), (0,)), ((), ())),
        precision=lax.Precision.HIGHEST,
        preferred_element_type=jnp.float32,
    )


def _pre_body(x_ref, wa_ref, wb_ref, b_ref, y_ref, rb_ref):
    xb = x_ref[...]
    y_ref[...] = _dot(xb, wa_ref[...])
    rb_ref[...] = _dot(xb, wb_ref[...]) + b_ref[...]


def _post_body(p_ref, h_ref, wa_ref, wb_ref, b_ref, o_ref):
    agg = p_ref[0] + p_ref[1]
    o_ref[...] = _dot(agg, wa_ref[...]) + _dot(h_ref[...], wb_ref[...]) + b_ref[...]


_row_spec = pl.BlockSpec((_BLK, D), lambda i: (i, 0))
_p_spec = pl.BlockSpec((NC, _BLK, D), lambda i: (0, i, 0))
_h_spec = pl.BlockSpec((1, _BLK, D), lambda i: (0, i, 0))
_w_spec = pl.BlockSpec((D, D), lambda i: (0, 0))
_b_spec = pl.BlockSpec((1, D), lambda i: (0, 0))
_f32 = jnp.float32


def _pre(x, wa, wb, b):
    return pl.pallas_call(
        _pre_body,
        grid=(NP // _BLK,),
        in_specs=[_row_spec, _w_spec, _w_spec, _b_spec],
        out_specs=[_row_spec, _row_spec],
        out_shape=[jax.ShapeDtypeStruct((NP, D), _f32)] * 2,
    )(x, wa, wb, b)


def _post(p, h, wa, wb, b):
    def h2d_body(p_ref, h_ref, wa_ref, wb_ref, b_ref, o_ref):
        _post_body(p_ref, h_ref[0], wa_ref, wb_ref, b_ref, o_ref)

    return pl.pallas_call(
        h2d_body,
        grid=(NP // _BLK,),
        in_specs=[_p_spec, _h_spec, _w_spec, _w_spec, _b_spec],
        out_specs=_row_spec,
        out_shape=jax.ShapeDtypeStruct((N, D), _f32),
    )(p, h, wa, wb, b)


def kernel(x, edge_index, W1_rel, b1, W1_root, W2_rel, b2, W2_root):
    src = edge_index[0]
    dst = edge_index[1]
    dstA = dst.reshape(NS, 4, NCH, 1, CHUNK)
    dstB = dst.reshape(NW, 2, NCH, 1, CHUNK)
    b1r = b1.reshape(1, D)
    b2r = b2.reshape(1, D)

    y1, r1b = _pre(x, W1_rel, W1_root, b1r)
    h, p2 = _sc_gnn(y1, r1b, src, dstA, dstB)
    return _post(p2, h, W2_rel, W2_root, b2r)
